# baseline (speedup 1.0000x reference)
import functools

import jax
import jax.numpy as jnp
from jax.experimental import pallas as pl
from jax.experimental.pallas import tpu as pltpu

G = 512
WIN = 64


def _body(se_ref, x_ref, batch_ref, batch_c_ref, w_ref, b_ref, out_ref,
          sums_ref, cnt_ref, ctx_ref):
    p = pl.program_id(0)
    i = pl.program_id(1)
    bvec = batch_ref[0]
    blk = x_ref.shape[0]
    g0 = jnp.minimum((se_ref[i, 0] // 8) * 8, G - WIN)
    ok = se_ref[i, 1] < g0 + WIN

    @pl.when((p == 0) & (i == 0))
    def _zero_acc():
        sums_ref[...] = jnp.zeros_like(sums_ref)
        cnt_ref[...] = jnp.zeros_like(cnt_ref)

    @pl.when((p == 0) & ok)
    def _pass_a_win():
        hit = g0 + jax.lax.broadcasted_iota(jnp.int32, (WIN, blk), 0) == bvec
        oh_t = hit.astype(jnp.bfloat16)
        xb = x_ref[...].astype(jnp.bfloat16)
        part = jnp.dot(oh_t, xb, preferred_element_type=jnp.float32)
        cnt_part = jnp.sum(hit.astype(jnp.float32), axis=1, keepdims=True)
        sums_ref[pl.ds(g0, WIN), :] += part
        cnt_ref[pl.ds(g0, WIN), 0:1] += cnt_part

    @pl.when((p == 0) & jnp.logical_not(ok))
    def _pass_a_full():
        hit = jax.lax.broadcasted_iota(jnp.int32, (G, blk), 0) == bvec
        oh_t = hit.astype(jnp.bfloat16)
        xb = x_ref[...].astype(jnp.bfloat16)
        part = jnp.dot(oh_t, xb, preferred_element_type=jnp.float32)
        cnt_part = jnp.sum(hit.astype(jnp.float32), axis=1, keepdims=True)
        sums_ref[...] += part
        cnt_ref[:, 0:1] += cnt_part

    @pl.when((p == 1) & (i == 0))
    def _mk_ctx():
        inv = 1.0 / jnp.maximum(cnt_ref[:, 0:1], 1.0)
        mean = sums_ref[...] * inv
        h = jax.lax.dot_general(
            mean, w_ref[...], (((1,), (1,)), ((), ())),
            preferred_element_type=jnp.float32,
        )
        ctx_ref[...] = jnp.tanh(h + b_ref[...])
        out_ref[...] = jnp.zeros_like(out_ref)

    @pl.when((p == 1) & ok)
    def _pass_b_win():
        bcol = batch_c_ref[0]
        hit_bw = g0 + jax.lax.broadcasted_iota(jnp.int32, (blk, WIN), 1) == bcol
        xb = x_ref[...].astype(jnp.bfloat16)
        ctx_win = ctx_ref[pl.ds(g0, WIN), :].astype(jnp.bfloat16)
        y = jax.lax.dot_general(
            xb, ctx_win, (((1,), (1,)), ((), ())),
            preferred_element_type=jnp.float32,
        )
        logits = jnp.sum(jnp.where(hit_bw, y, 0.0), axis=1, keepdims=True)
        score = jax.nn.sigmoid(logits)
        oh_s = jnp.where(hit_bw, score, 0.0).astype(jnp.bfloat16)
        part = jax.lax.dot_general(
            oh_s, xb, (((0,), (0,)), ((), ())),
            preferred_element_type=jnp.float32,
        )
        out_ref[pl.ds(g0, WIN), :] += part

    @pl.when((p == 1) & jnp.logical_not(ok))
    def _pass_b_full():
        hit = jax.lax.broadcasted_iota(jnp.int32, (G, blk), 0) == bvec
        oh_t = hit.astype(jnp.bfloat16)
        xb = x_ref[...]
        ctxn = jax.lax.dot_general(
            oh_t, ctx_ref[...].astype(jnp.bfloat16), (((0,), (0,)), ((), ())),
            preferred_element_type=jnp.float32,
        )
        logits = jnp.sum(xb * ctxn, axis=1, keepdims=True)
        score = jax.nn.sigmoid(logits)
        part = jnp.dot(oh_t, (score * xb).astype(jnp.bfloat16),
                       preferred_element_type=jnp.float32)
        out_ref[...] += part


@functools.partial(jax.jit, static_argnames=())
def kernel(x, batch, W, b):
    n, d = x.shape
    blk = 2000 if n % 2000 == 0 else 8
    nb = n // blk
    batch32 = batch.astype(jnp.int32)
    br = batch32.reshape(nb, blk)
    se = jnp.stack([br[:, 0], br[:, -1]], axis=1)
    batch_r = br.reshape(nb, 1, blk)
    batch_c = br.reshape(nb, blk, 1)
    b2 = b.reshape(1, d)
    return pl.pallas_call(
        _body,
        grid=(2, nb),
        in_specs=[
            pl.BlockSpec(memory_space=pltpu.SMEM),
            pl.BlockSpec((blk, d), lambda p, i: (i, 0)),
            pl.BlockSpec((1, 1, blk), lambda p, i: (i, 0, 0)),
            pl.BlockSpec((1, blk, 1), lambda p, i: (i, 0, 0)),
            pl.BlockSpec((d, d), lambda p, i: (0, 0)),
            pl.BlockSpec((1, d), lambda p, i: (0, 0)),
        ],
        out_specs=pl.BlockSpec((G, d), lambda p, i: (0, 0)),
        out_shape=jax.ShapeDtypeStruct((G, d), jnp.float32),
        scratch_shapes=[
            pltpu.VMEM((G, d), jnp.float32),
            pltpu.VMEM((G, 128), jnp.float32),
            pltpu.VMEM((G, d), jnp.float32),
        ],
    )(se, x, batch_r, batch_c, W, b2)


# speedup vs baseline: 1.7850x; 1.7850x over previous
import functools

import jax
import jax.numpy as jnp
from jax.experimental import pallas as pl
from jax.experimental.pallas import tpu as pltpu

G = 512
WIN = 64


def _body(se_ref, x_ref, batch_ref, w_ref, b_ref, out_ref,
          sums_ref, cnt_ref, ctx_ref):
    p = pl.program_id(0)
    i = pl.program_id(1)
    bvec = batch_ref[0]
    blk = x_ref.shape[0]
    g0 = jnp.minimum((se_ref[i, 0] // 8) * 8, G - WIN)
    ok = se_ref[i, 1] < g0 + WIN

    @pl.when((p == 0) & (i == 0))
    def _zero_acc():
        sums_ref[...] = jnp.zeros_like(sums_ref)
        cnt_ref[...] = jnp.zeros_like(cnt_ref)

    @pl.when((p == 0) & ok)
    def _pass_a_win():
        hit = g0 + jax.lax.broadcasted_iota(jnp.int32, (WIN, blk), 0) == bvec
        oh_t = hit.astype(jnp.bfloat16)
        xb = x_ref[...].astype(jnp.bfloat16)
        part = jnp.dot(oh_t, xb, preferred_element_type=jnp.float32)
        cnt_part = jnp.sum(hit.astype(jnp.float32), axis=1, keepdims=True)
        sums_ref[pl.ds(g0, WIN), :] += part
        cnt_ref[pl.ds(g0, WIN), 0:1] += cnt_part

    @pl.when((p == 0) & jnp.logical_not(ok))
    def _pass_a_full():
        hit = jax.lax.broadcasted_iota(jnp.int32, (G, blk), 0) == bvec
        oh_t = hit.astype(jnp.bfloat16)
        xb = x_ref[...].astype(jnp.bfloat16)
        part = jnp.dot(oh_t, xb, preferred_element_type=jnp.float32)
        cnt_part = jnp.sum(hit.astype(jnp.float32), axis=1, keepdims=True)
        sums_ref[...] += part
        cnt_ref[:, 0:1] += cnt_part

    @pl.when((p == 1) & (i == 0))
    def _mk_ctx():
        inv = 1.0 / jnp.maximum(cnt_ref[:, 0:1], 1.0)
        mean = sums_ref[...] * inv
        h = jax.lax.dot_general(
            mean, w_ref[...], (((1,), (1,)), ((), ())),
            preferred_element_type=jnp.float32,
        )
        ctx_ref[...] = jnp.tanh(h + b_ref[...])
        out_ref[...] = jnp.zeros_like(out_ref)

    @pl.when((p == 1) & ok)
    def _pass_b_win():
        hit = g0 + jax.lax.broadcasted_iota(jnp.int32, (WIN, blk), 0) == bvec
        oh_t = hit.astype(jnp.bfloat16)
        xb = x_ref[...]
        ctx_win = ctx_ref[pl.ds(g0, WIN), :].astype(jnp.bfloat16)
        ctxn = jax.lax.dot_general(
            oh_t, ctx_win, (((0,), (0,)), ((), ())),
            preferred_element_type=jnp.float32,
        )
        logits = jnp.sum(xb * ctxn, axis=1, keepdims=True)
        score = jax.nn.sigmoid(logits)
        part = jnp.dot(oh_t, (score * xb).astype(jnp.bfloat16),
                       preferred_element_type=jnp.float32)
        out_ref[pl.ds(g0, WIN), :] += part

    @pl.when((p == 1) & jnp.logical_not(ok))
    def _pass_b_full():
        hit = jax.lax.broadcasted_iota(jnp.int32, (G, blk), 0) == bvec
        oh_t = hit.astype(jnp.bfloat16)
        xb = x_ref[...]
        ctxn = jax.lax.dot_general(
            oh_t, ctx_ref[...].astype(jnp.bfloat16), (((0,), (0,)), ((), ())),
            preferred_element_type=jnp.float32,
        )
        logits = jnp.sum(xb * ctxn, axis=1, keepdims=True)
        score = jax.nn.sigmoid(logits)
        part = jnp.dot(oh_t, (score * xb).astype(jnp.bfloat16),
                       preferred_element_type=jnp.float32)
        out_ref[...] += part


@functools.partial(jax.jit, static_argnames=())
def kernel(x, batch, W, b):
    n, d = x.shape
    blk = 4000 if n % 4000 == 0 else (2000 if n % 2000 == 0 else 8)
    nb = n // blk
    batch32 = batch.astype(jnp.int32)
    br = batch32.reshape(nb, blk)
    se = jnp.stack([br[:, 0], br[:, -1]], axis=1)
    batch_r = br.reshape(nb, 1, blk)
    b2 = b.reshape(1, d)
    return pl.pallas_call(
        _body,
        grid=(2, nb),
        in_specs=[
            pl.BlockSpec(memory_space=pltpu.SMEM),
            pl.BlockSpec((blk, d), lambda p, i: (i, 0)),
            pl.BlockSpec((1, 1, blk), lambda p, i: (i, 0, 0)),
            pl.BlockSpec((d, d), lambda p, i: (0, 0)),
            pl.BlockSpec((1, d), lambda p, i: (0, 0)),
        ],
        out_specs=pl.BlockSpec((G, d), lambda p, i: (0, 0)),
        out_shape=jax.ShapeDtypeStruct((G, d), jnp.float32),
        scratch_shapes=[
            pltpu.VMEM((G, d), jnp.float32),
            pltpu.VMEM((G, 128), jnp.float32),
            pltpu.VMEM((G, d), jnp.float32),
        ],
    )(se, x, batch_r, W, b2)


# speedup vs baseline: 1.8298x; 1.0251x over previous
import functools

import jax
import jax.numpy as jnp
from jax.experimental import pallas as pl
from jax.experimental.pallas import tpu as pltpu

G = 512
WIN = 64


def _body(se_ref, x_ref, batch_ref, w_ref, b_ref, out_ref,
          sums_ref, cnt_ref, ctx_ref):
    p = pl.program_id(0)
    i = pl.program_id(1)
    bvec = batch_ref[0]
    blk = x_ref.shape[0]
    g0 = jnp.minimum((se_ref[i, 0] // 8) * 8, G - WIN)
    ok = se_ref[i, 1] < g0 + WIN

    @pl.when((p == 0) & (i == 0))
    def _zero_acc():
        sums_ref[...] = jnp.zeros_like(sums_ref)
        cnt_ref[...] = jnp.zeros_like(cnt_ref)

    @pl.when((p == 0) & ok)
    def _pass_a_win():
        hit = g0 + jax.lax.broadcasted_iota(jnp.int32, (WIN, blk), 0) == bvec
        oh_t = hit.astype(jnp.bfloat16)
        xb = x_ref[...].astype(jnp.bfloat16)
        part = jnp.dot(oh_t, xb, preferred_element_type=jnp.float32)
        cnt_part = jnp.sum(hit.astype(jnp.float32), axis=1, keepdims=True)
        sums_ref[pl.ds(g0, WIN), :] += part
        cnt_ref[pl.ds(g0, WIN), 0:1] += cnt_part

    @pl.when((p == 0) & jnp.logical_not(ok))
    def _pass_a_full():
        hit = jax.lax.broadcasted_iota(jnp.int32, (G, blk), 0) == bvec
        oh_t = hit.astype(jnp.bfloat16)
        xb = x_ref[...].astype(jnp.bfloat16)
        part = jnp.dot(oh_t, xb, preferred_element_type=jnp.float32)
        cnt_part = jnp.sum(hit.astype(jnp.float32), axis=1, keepdims=True)
        sums_ref[...] += part
        cnt_ref[:, 0:1] += cnt_part

    @pl.when((p == 1) & (i == 0))
    def _mk_ctx():
        inv = 1.0 / jnp.maximum(cnt_ref[:, 0:1], 1.0)
        mean = sums_ref[...] * inv
        h = jax.lax.dot_general(
            mean, w_ref[...], (((1,), (1,)), ((), ())),
            preferred_element_type=jnp.float32,
        )
        ctx_ref[...] = jnp.tanh(h + b_ref[...])
        out_ref[...] = jnp.zeros_like(out_ref)

    @pl.when((p == 1) & ok)
    def _pass_b_win():
        hit = g0 + jax.lax.broadcasted_iota(jnp.int32, (WIN, blk), 0) == bvec
        oh_t = hit.astype(jnp.bfloat16)
        xb = x_ref[...]
        ctx_win = ctx_ref[pl.ds(g0, WIN), :].astype(jnp.bfloat16)
        ctxn = jax.lax.dot_general(
            oh_t, ctx_win, (((0,), (0,)), ((), ())),
            preferred_element_type=jnp.float32,
        )
        logits = jnp.sum(xb * ctxn, axis=1, keepdims=True)
        score = jax.nn.sigmoid(logits)
        part = jnp.dot(oh_t, (score * xb).astype(jnp.bfloat16),
                       preferred_element_type=jnp.float32)
        out_ref[pl.ds(g0, WIN), :] += part

    @pl.when((p == 1) & jnp.logical_not(ok))
    def _pass_b_full():
        hit = jax.lax.broadcasted_iota(jnp.int32, (G, blk), 0) == bvec
        oh_t = hit.astype(jnp.bfloat16)
        xb = x_ref[...]
        ctxn = jax.lax.dot_general(
            oh_t, ctx_ref[...].astype(jnp.bfloat16), (((0,), (0,)), ((), ())),
            preferred_element_type=jnp.float32,
        )
        logits = jnp.sum(xb * ctxn, axis=1, keepdims=True)
        score = jax.nn.sigmoid(logits)
        part = jnp.dot(oh_t, (score * xb).astype(jnp.bfloat16),
                       preferred_element_type=jnp.float32)
        out_ref[...] += part


@functools.partial(jax.jit, static_argnames=())
def kernel(x, batch, W, b):
    n, d = x.shape
    blk = 5000 if n % 5000 == 0 else (2000 if n % 2000 == 0 else 8)
    nb = n // blk
    batch32 = batch.astype(jnp.int32)
    br = batch32.reshape(nb, blk)
    se = jnp.stack([br[:, 0], br[:, -1]], axis=1)
    batch_r = br.reshape(nb, 1, blk)
    b2 = b.reshape(1, d)
    return pl.pallas_call(
        _body,
        grid=(2, nb),
        in_specs=[
            pl.BlockSpec(memory_space=pltpu.SMEM),
            pl.BlockSpec((blk, d), lambda p, i: (i, 0)),
            pl.BlockSpec((1, 1, blk), lambda p, i: (i, 0, 0)),
            pl.BlockSpec((d, d), lambda p, i: (0, 0)),
            pl.BlockSpec((1, d), lambda p, i: (0, 0)),
        ],
        out_specs=pl.BlockSpec((G, d), lambda p, i: (0, 0)),
        out_shape=jax.ShapeDtypeStruct((G, d), jnp.float32),
        scratch_shapes=[
            pltpu.VMEM((G, d), jnp.float32),
            pltpu.VMEM((G, 128), jnp.float32),
            pltpu.VMEM((G, d), jnp.float32),
        ],
    )(se, x, batch_r, W, b2)


# speedup vs baseline: 1.8359x; 1.0034x over previous
import functools

import jax
import jax.numpy as jnp
from jax.experimental import pallas as pl
from jax.experimental.pallas import tpu as pltpu

G = 512
WIN = 64


def _body(se_ref, x_ref, batch_ref, w_ref, b_ref, out_ref,
          sums_ref, cnt_ref, ctx_ref):
    p = pl.program_id(0)
    i = pl.program_id(1)
    bvec = batch_ref[0]
    blk = x_ref.shape[0]
    g0 = jnp.minimum((se_ref[i, 0] // 8) * 8, G - WIN)
    ok = se_ref[i, 1] < g0 + WIN

    @pl.when((p == 0) & (i == 0))
    def _zero_acc():
        sums_ref[...] = jnp.zeros_like(sums_ref)
        cnt_ref[...] = jnp.zeros_like(cnt_ref)

    @pl.when((p == 0) & ok)
    def _pass_a_win():
        hit = g0 + jax.lax.broadcasted_iota(jnp.int32, (WIN, blk), 0) == bvec
        oh_t = hit.astype(jnp.bfloat16)
        xb = x_ref[...].astype(jnp.bfloat16)
        part = jnp.dot(oh_t, xb, preferred_element_type=jnp.float32)
        cnt_part = jnp.sum(hit.astype(jnp.float32), axis=1, keepdims=True)
        sums_ref[pl.ds(g0, WIN), :] += part
        cnt_ref[pl.ds(g0, WIN), 0:1] += cnt_part

    @pl.when((p == 0) & jnp.logical_not(ok))
    def _pass_a_full():
        hit = jax.lax.broadcasted_iota(jnp.int32, (G, blk), 0) == bvec
        oh_t = hit.astype(jnp.bfloat16)
        xb = x_ref[...].astype(jnp.bfloat16)
        part = jnp.dot(oh_t, xb, preferred_element_type=jnp.float32)
        cnt_part = jnp.sum(hit.astype(jnp.float32), axis=1, keepdims=True)
        sums_ref[...] += part
        cnt_ref[:, 0:1] += cnt_part

    @pl.when((p == 1) & (i == 0))
    def _mk_ctx():
        inv = 1.0 / jnp.maximum(cnt_ref[:, 0:1], 1.0)
        mean = sums_ref[...] * inv
        h = jax.lax.dot_general(
            mean, w_ref[...], (((1,), (1,)), ((), ())),
            preferred_element_type=jnp.float32,
        )
        ctx_ref[...] = jnp.tanh(h + b_ref[...])
        out_ref[...] = jnp.zeros_like(out_ref)

    @pl.when((p == 1) & ok)
    def _pass_b_win():
        hit = g0 + jax.lax.broadcasted_iota(jnp.int32, (WIN, blk), 0) == bvec
        oh_t = hit.astype(jnp.bfloat16)
        xb = x_ref[...]
        ctx_win = ctx_ref[pl.ds(g0, WIN), :].astype(jnp.bfloat16)
        ctxn = jax.lax.dot_general(
            oh_t, ctx_win, (((0,), (0,)), ((), ())),
            preferred_element_type=jnp.float32,
        )
        logits = jnp.sum(xb * ctxn, axis=1, keepdims=True)
        score = jax.nn.sigmoid(logits)
        oh_s = oh_t * score.astype(jnp.bfloat16).T
        part = jnp.dot(oh_s, xb.astype(jnp.bfloat16),
                       preferred_element_type=jnp.float32)
        out_ref[pl.ds(g0, WIN), :] += part

    @pl.when((p == 1) & jnp.logical_not(ok))
    def _pass_b_full():
        hit = jax.lax.broadcasted_iota(jnp.int32, (G, blk), 0) == bvec
        oh_t = hit.astype(jnp.bfloat16)
        xb = x_ref[...]
        ctxn = jax.lax.dot_general(
            oh_t, ctx_ref[...].astype(jnp.bfloat16), (((0,), (0,)), ((), ())),
            preferred_element_type=jnp.float32,
        )
        logits = jnp.sum(xb * ctxn, axis=1, keepdims=True)
        score = jax.nn.sigmoid(logits)
        part = jnp.dot(oh_t, (score * xb).astype(jnp.bfloat16),
                       preferred_element_type=jnp.float32)
        out_ref[...] += part


@functools.partial(jax.jit, static_argnames=())
def kernel(x, batch, W, b):
    n, d = x.shape
    blk = 5000 if n % 5000 == 0 else (2000 if n % 2000 == 0 else 8)
    nb = n // blk
    batch32 = batch.astype(jnp.int32)
    br = batch32.reshape(nb, blk)
    se = jnp.stack([br[:, 0], br[:, -1]], axis=1)
    batch_r = br.reshape(nb, 1, blk)
    b2 = b.reshape(1, d)
    return pl.pallas_call(
        _body,
        grid=(2, nb),
        in_specs=[
            pl.BlockSpec(memory_space=pltpu.SMEM),
            pl.BlockSpec((blk, d), lambda p, i: (i, 0)),
            pl.BlockSpec((1, 1, blk), lambda p, i: (i, 0, 0)),
            pl.BlockSpec((d, d), lambda p, i: (0, 0)),
            pl.BlockSpec((1, d), lambda p, i: (0, 0)),
        ],
        out_specs=pl.BlockSpec((G, d), lambda p, i: (0, 0)),
        out_shape=jax.ShapeDtypeStruct((G, d), jnp.float32),
        scratch_shapes=[
            pltpu.VMEM((G, d), jnp.float32),
            pltpu.VMEM((G, 128), jnp.float32),
            pltpu.VMEM((G, d), jnp.float32),
        ],
    )(se, x, batch_r, W, b2)
